# Initial kernel scaffold; baseline (speedup 1.0000x reference)
import jax
import jax.numpy as jnp
from jax.experimental import pallas as pl


def kernel(x, embeddings_table):
    raise NotImplementedError("write your pallas kernel here")


# speedup vs baseline: 2.5626x; 2.5626x over previous
import jax
import jax.numpy as jnp
from jax import lax
from jax.experimental import pallas as pl
from jax.experimental.pallas import tpu as pltpu
from jax.experimental.pallas import tpu_sc as plsc

D_MODEL = 256
TABLE_ROWS = 129
SEQ_LEN = 512
S_LEN = 2 * SEQ_LEN
REP0_ROWS = SEQ_LEN - 64
REP1_ROWS = SEQ_LEN - 65
LANES = 16
ROWS_PER_WORKER = SEQ_LEN // 32


def _replicate_row(rep_vmem, n_rows):
    row = [rep_vmem[0, pl.ds(LANES * l, LANES)] for l in range(D_MODEL // LANES)]

    def body(n, carry):
        for l in range(D_MODEL // LANES):
            rep_vmem[n, pl.ds(LANES * l, LANES)] = row[l]
        return carry

    lax.fori_loop(1, n_rows, body, 0)


def _sc_body(table_hbm, out_hbm, s_spmem, rep_vmem, sem):
    c = lax.axis_index("c")
    s = lax.axis_index("s")

    @pl.when(s == 0)
    def _():
        pltpu.sync_copy(table_hbm.at[pl.ds(0, 1)], rep_vmem.at[pl.ds(0, 1)])
        _replicate_row(rep_vmem, REP0_ROWS)
        pltpu.sync_copy(rep_vmem.at[pl.ds(0, REP0_ROWS)],
                        s_spmem.at[pl.ds(0, REP0_ROWS)])

    @pl.when(s == 1)
    def _():
        pltpu.sync_copy(table_hbm.at[pl.ds(TABLE_ROWS - 1, 1)],
                        rep_vmem.at[pl.ds(0, 1)])
        _replicate_row(rep_vmem, REP1_ROWS)
        pltpu.sync_copy(rep_vmem.at[pl.ds(0, REP1_ROWS)],
                        s_spmem.at[pl.ds(REP0_ROWS + TABLE_ROWS, REP1_ROWS)])

    @pl.when(s == 2)
    def _():
        pltpu.sync_copy(table_hbm, s_spmem.at[pl.ds(REP0_ROWS, TABLE_ROWS)])

    plsc.subcore_barrier()

    wid = c * 16 + s
    copies = []
    for k in range(ROWS_PER_WORKER):
        i = wid * ROWS_PER_WORKER + k
        copies.append(
            pltpu.async_copy(s_spmem.at[pl.ds(SEQ_LEN - i, SEQ_LEN)],
                             out_hbm.at[i], sem))
    for cp in copies:
        cp.wait()


def kernel(x, embeddings_table):
    mesh = plsc.VectorSubcoreMesh(core_axis_name="c", subcore_axis_name="s")
    run = pl.kernel(
        _sc_body,
        out_type=jax.ShapeDtypeStruct((SEQ_LEN, SEQ_LEN, D_MODEL), jnp.float32),
        mesh=mesh,
        compiler_params=pltpu.CompilerParams(use_tc_tiling_on_sc=False),
        scratch_types=[
            pltpu.VMEM_SHARED((S_LEN, D_MODEL), jnp.float32),
            pltpu.VMEM((REP0_ROWS, D_MODEL), jnp.float32),
            pltpu.SemaphoreType.DMA,
        ],
    )
    out = run(embeddings_table.astype(jnp.float32))
    return out.astype(x.dtype)


# speedup vs baseline: 6.8864x; 2.6873x over previous
import jax
import jax.numpy as jnp
from jax import lax
from jax.experimental import pallas as pl
from jax.experimental.pallas import tpu as pltpu
from jax.experimental.pallas import tpu_sc as plsc

D_MODEL = 256
TABLE_ROWS = 129
SEQ_LEN = 512

T_LEN = 1016
N_COPIES = 8
REP_LEN = 440
MID_LO, MID_HI = 440, 576
MID_LEN = MID_HI - MID_LO
PARTS_MIDS = 2 * REP_LEN
PARTS_LEN = 2048
ROWS_PER_WORKER = SEQ_LEN // 32


def _build_parts(tbl_ref, parts_ref):
    t = tbl_ref[...]
    t0 = t[0:1]
    t128 = t[TABLE_ROWS - 1:TABLE_ROWS]
    pieces = [
        jnp.broadcast_to(t0, (REP_LEN, D_MODEL)),
        jnp.broadcast_to(t128, (REP_LEN, D_MODEL)),
    ]
    for r in range(1, N_COPIES + 1):
        if 8 - r:
            pieces.append(jnp.broadcast_to(t0, (8 - r, D_MODEL)))
        pieces.append(t)
        if r - 1:
            pieces.append(jnp.broadcast_to(t128, (r - 1, D_MODEL)))
    used = PARTS_MIDS + N_COPIES * MID_LEN
    pieces.append(jnp.broadcast_to(t128, (PARTS_LEN - used, D_MODEL)))
    parts_ref[...] = jnp.concatenate(pieces, axis=0)


def _sc_body(parts_hbm, out_hbm, t_all, sem):
    c = lax.axis_index("c")
    s = lax.axis_index("s")

    @pl.when(s < N_COPIES)
    def _():
        base = pl.multiple_of(s * T_LEN, 8)
        pltpu.sync_copy(parts_hbm.at[pl.ds(0, REP_LEN)],
                        t_all.at[pl.ds(base, REP_LEN)])
        src_mid = pl.multiple_of(PARTS_MIDS + s * MID_LEN, 8)
        dst_mid = pl.multiple_of(s * T_LEN + MID_LO, 8)
        pltpu.sync_copy(parts_hbm.at[pl.ds(src_mid, MID_LEN)],
                        t_all.at[pl.ds(dst_mid, MID_LEN)])

    @pl.when(s >= N_COPIES)
    def _():
        dst = pl.multiple_of((s - N_COPIES) * T_LEN + MID_HI, 8)
        pltpu.sync_copy(parts_hbm.at[pl.ds(REP_LEN, REP_LEN)],
                        t_all.at[pl.ds(dst, REP_LEN)])

    plsc.subcore_barrier()

    w = c * 16 + s
    copies = []
    for k in range(ROWS_PER_WORKER):
        i = 16 * w + k
        r = N_COPIES - (k % 8)
        a0 = 504 - 16 * w - 8 * (k // 8)
        off = pl.multiple_of((r - 1) * T_LEN + a0, 8)
        copies.append(
            pltpu.async_copy(t_all.at[pl.ds(off, SEQ_LEN)],
                             out_hbm.at[i], sem))
    for cp in copies:
        cp.wait()


def kernel(x, embeddings_table):
    table = embeddings_table.astype(jnp.float32)
    parts = pl.pallas_call(
        _build_parts,
        out_shape=jax.ShapeDtypeStruct((PARTS_LEN, D_MODEL), jnp.float32),
    )(table)

    mesh = plsc.VectorSubcoreMesh(core_axis_name="c", subcore_axis_name="s")
    run = pl.kernel(
        _sc_body,
        out_type=jax.ShapeDtypeStruct((SEQ_LEN, SEQ_LEN, D_MODEL), jnp.float32),
        mesh=mesh,
        scratch_types=[
            pltpu.VMEM_SHARED((N_COPIES * T_LEN, D_MODEL), jnp.float32),
            pltpu.SemaphoreType.DMA,
        ],
    )
    out = run(parts)
    return out.astype(x.dtype)


# speedup vs baseline: 6.9353x; 1.0071x over previous
import jax
import jax.numpy as jnp
from jax import lax
from jax.experimental import pallas as pl
from jax.experimental.pallas import tpu as pltpu
from jax.experimental.pallas import tpu_sc as plsc

D_MODEL = 256
TABLE_ROWS = 129
SEQ_LEN = 512

T_LEN = 1016
N_COPIES = 8
REP_LEN = 440
MID_LO, MID_HI = 440, 576
MID_LEN = MID_HI - MID_LO
MID_SPLIT = 64
PARTS_MIDS = 2 * REP_LEN
PARTS_LEN = 2048
ROWS_PER_WORKER = SEQ_LEN // 32


def _build_parts(tbl_ref, parts_ref):
    t = tbl_ref[...]
    t0 = t[0:1]
    t128 = t[TABLE_ROWS - 1:TABLE_ROWS]
    pieces = [
        jnp.broadcast_to(t0, (REP_LEN, D_MODEL)),
        jnp.broadcast_to(t128, (REP_LEN, D_MODEL)),
    ]
    for r in range(1, N_COPIES + 1):
        if 8 - r:
            pieces.append(jnp.broadcast_to(t0, (8 - r, D_MODEL)))
        pieces.append(t)
        if r - 1:
            pieces.append(jnp.broadcast_to(t128, (r - 1, D_MODEL)))
    used = PARTS_MIDS + N_COPIES * MID_LEN
    pieces.append(jnp.broadcast_to(t128, (PARTS_LEN - used, D_MODEL)))
    parts_ref[...] = jnp.concatenate(pieces, axis=0)


def _sc_body(parts_hbm, out_hbm, t_all, sem):
    c = lax.axis_index("c")
    s = lax.axis_index("s")

    @pl.when(s < N_COPIES)
    def _():
        base = pl.multiple_of(s * T_LEN, 8)
        cp0 = pltpu.async_copy(parts_hbm.at[pl.ds(0, REP_LEN)],
                               t_all.at[pl.ds(base, REP_LEN)], sem)
        src_mid = pl.multiple_of(PARTS_MIDS + s * MID_LEN, 8)
        dst_mid = pl.multiple_of(s * T_LEN + MID_LO, 8)
        cp1 = pltpu.async_copy(parts_hbm.at[pl.ds(src_mid, MID_SPLIT)],
                               t_all.at[pl.ds(dst_mid, MID_SPLIT)], sem)
        cp0.wait()
        cp1.wait()

    @pl.when(s >= N_COPIES)
    def _():
        s8 = s - N_COPIES
        src_mid = pl.multiple_of(PARTS_MIDS + s8 * MID_LEN + MID_SPLIT, 8)
        dst_mid = pl.multiple_of(s8 * T_LEN + MID_LO + MID_SPLIT, 8)
        cp0 = pltpu.async_copy(
            parts_hbm.at[pl.ds(src_mid, MID_LEN - MID_SPLIT)],
            t_all.at[pl.ds(dst_mid, MID_LEN - MID_SPLIT)], sem)
        dst = pl.multiple_of(s8 * T_LEN + MID_HI, 8)
        cp1 = pltpu.async_copy(parts_hbm.at[pl.ds(REP_LEN, REP_LEN)],
                               t_all.at[pl.ds(dst, REP_LEN)], sem)
        cp0.wait()
        cp1.wait()

    plsc.subcore_barrier()

    w = c * 16 + s
    copies = []
    for k in range(ROWS_PER_WORKER):
        i = 16 * w + k
        r = N_COPIES - (k % 8)
        a0 = 504 - 16 * w - 8 * (k // 8)
        off = pl.multiple_of((r - 1) * T_LEN + a0, 8)
        copies.append(
            pltpu.async_copy(t_all.at[pl.ds(off, SEQ_LEN)],
                             out_hbm.at[i], sem))
    for cp in copies:
        cp.wait()


def kernel(x, embeddings_table):
    table = embeddings_table.astype(jnp.float32)
    parts = pl.pallas_call(
        _build_parts,
        out_shape=jax.ShapeDtypeStruct((PARTS_LEN, D_MODEL), jnp.float32),
    )(table)

    mesh = plsc.VectorSubcoreMesh(core_axis_name="c", subcore_axis_name="s")
    run = pl.kernel(
        _sc_body,
        out_type=jax.ShapeDtypeStruct((SEQ_LEN, SEQ_LEN, D_MODEL), jnp.float32),
        mesh=mesh,
        scratch_types=[
            pltpu.VMEM_SHARED((N_COPIES * T_LEN, D_MODEL), jnp.float32),
            pltpu.SemaphoreType.DMA,
        ],
    )
    out = run(parts)
    return out.astype(x.dtype)
